# baseline (speedup 1.0000x reference)
import functools

import jax
import jax.numpy as jnp
from jax import lax
from jax.experimental import pallas as pl
from jax.experimental.pallas import tpu as pltpu
from jax.experimental.pallas import tpu_sc as plsc

_BATCH = 1024
_HIST = 20
_EMBED_DIM = 64
_N = _BATCH * _HIST

_INFO = plsc.get_sparse_core_info()
_NC = _INFO.num_cores
_NS = _INFO.num_subcores
_NW = _NC * _NS
_PER_W = _N // _NW
_CHUNK = 128
_NCHUNK = _PER_W // _CHUNK


def _make_gather(vocab: int, d: int):
    mesh = plsc.VectorSubcoreMesh(core_axis_name="c", subcore_axis_name="s")

    @functools.partial(
        pl.kernel,
        mesh=mesh,
        out_type=jax.ShapeDtypeStruct((_N, d), jnp.float32),
        compiler_params=pltpu.CompilerParams(use_tc_tiling_on_sc=False),
        scratch_types=[
            pltpu.VMEM((_PER_W,), jnp.int32),
            pltpu.VMEM((_PER_W, d), jnp.float32),
            pltpu.SemaphoreType.DMA,
        ],
    )
    def gather(table_hbm, idx_hbm, out_hbm, idx_v, rows_v, sem):
        wid = lax.axis_index("s") * _NC + lax.axis_index("c")
        base = wid * _PER_W
        pltpu.sync_copy(idx_hbm.at[pl.ds(base, _PER_W)], idx_v)
        copies = []
        for c in range(_NCHUNK):
            copies.append(
                pltpu.async_copy(
                    table_hbm.at[idx_v.at[pl.ds(c * _CHUNK, _CHUNK)]],
                    rows_v.at[pl.ds(c * _CHUNK, _CHUNK)],
                    sem,
                )
            )
        for cp in copies:
            cp.wait()
        pltpu.sync_copy(rows_v, out_hbm.at[pl.ds(base, _PER_W)])

    return gather


def kernel(input, weight):
    vocab, d = weight.shape
    idx = input.reshape(_N).astype(jnp.int32)
    out = _make_gather(vocab, d)(weight, idx)
    return out.reshape(_BATCH, _HIST, d)


# speedup vs baseline: 1.9884x; 1.9884x over previous
import functools

import jax
import jax.numpy as jnp
from jax import lax
from jax.experimental import pallas as pl
from jax.experimental.pallas import tpu as pltpu
from jax.experimental.pallas import tpu_sc as plsc

_BATCH = 1024
_HIST = 20
_D = 64
_N = _BATCH * _HIST
_V = 1000000
_NW = 32
_LANE = 128
_CPC = 2
_CHW = _CPC * _LANE
_NCH = (_V + _CHW - 1) // _CHW
_NOUT = _N + 8
_DUMP = _N


def _make_sweep(V=_V, N=_N):
    mesh = plsc.VectorSubcoreMesh(core_axis_name="c", subcore_axis_name="s")
    nch = (V + _CHW - 1) // _CHW
    tcols = (V + _LANE - 1) // _LANE
    nout = N + 8
    dump = N
    nsub = N // 2560

    @functools.partial(
        pl.kernel,
        mesh=mesh,
        out_type=jax.ShapeDtypeStruct((nout, _LANE), jnp.float32),
        compiler_params=pltpu.CompilerParams(
            use_tc_tiling_on_sc=True, needs_layout_passes=False
        ),
        scratch_types=[
            pltpu.VMEM((2560,), jnp.int32),
            pltpu.VMEM((N + 16,), jnp.int32),
            pltpu.VMEM((N + 16,), jnp.int32),
            pltpu.VMEM((2, _CPC, _D, _LANE), jnp.float32),
            pltpu.VMEM((128, _LANE), jnp.float32),
            pltpu.VMEM((128,), jnp.int32),
            pltpu.VMEM((32,), jnp.int32),
            pltpu.VMEM((32,), jnp.int32),
            pltpu.SemaphoreType.DMA((2,)),
            pltpu.SemaphoreType.DMA,
        ],
    )
    def k(wt, idx, out, idx_st, vlist, jlist, slab, rows, jb, minv, minj,
          sems, ssem):
        t = lax.axis_index("s") * 2 + lax.axis_index("c")
        iota = lax.iota(jnp.int32, 16)
        lane0 = iota == 0
        nl = jnp.where(t < nch - (nch // _NW) * _NW, nch // _NW + 1,
                       nch // _NW)

        wp = jnp.int32(0)
        for sub in range(nsub):
            pltpu.sync_copy(idx.at[pl.ds(sub * 2560, 2560)], idx_st)

            def fbody(r, wp, sub=sub):
                vv = idx_st[pl.ds(r * 16, 16)]
                jj = iota + (sub * 2560) + r * 16
                m = ((vv >> 8) & (_NW - 1)) == t
                plsc.store_compressed(vlist.at[pl.ds(wp, 16)], vv, mask=m)
                plsc.store_compressed(jlist.at[pl.ds(wp, 16)], jj, mask=m)
                return wp + plsc.all_reduce_population_count(m)[0]

            wp = lax.fori_loop(0, 160, fbody, wp)
        vlist[pl.ds(wp, 16)] = jnp.full((16,), -1, jnp.int32)
        jlist[pl.ds(wp, 16)] = jnp.full((16,), dump, jnp.int32)
        nvr = (wp + 15) >> 4

        for g8 in range(8):
            jb[pl.ds(g8 * 16, 16)] = jnp.full((16,), dump, jnp.int32)

        slab5 = slab.reshape(2, _CPC, 8, 8, _LANE)
        a_vec = [(iota + 16 * gi) >> 3 for gi in range(4)]
        s_vec = [(iota + 16 * gi) & 7 for gi in range(4)]

        def fetch(g, buf):
            for c in range(_CPC):
                col = jnp.minimum(g * _CPC + c, tcols - 1)
                col0 = pl.multiple_of(col * _LANE, _LANE)
                pltpu.async_copy(
                    wt.at[:, pl.ds(col0, _LANE)], slab.at[buf, c], sems.at[buf]
                )

        def wait_slab(buf):
            for c in range(_CPC):
                pltpu.make_async_copy(
                    wt.at[:, pl.ds(0, _LANE)], slab.at[buf, c], sems.at[buf]
                ).wait()

        def flush():
            pltpu.async_copy(rows, out.at[jb], ssem).wait()
            for g8 in range(8):
                jb[pl.ds(g8 * 16, 16)] = jnp.full((16,), dump, jnp.int32)

        fetch(t, 0)

        def chunk_body(l, slot):
            buf = l & 1
            g = l * _NW + t

            @pl.when(l + 1 < nl)
            def _():
                fetch((l + 1) * _NW + t, buf ^ 1)

            wait_slab(buf)

            def rbody(r, slot):
                vv = vlist[pl.ds(r * 16, 16)]
                jj = jlist[pl.ds(r * 16, 16)]
                m = (vv >> 8) == g
                cnt = plsc.all_reduce_population_count(m)[0]
                plsc.store_compressed(minv.at[pl.ds(0, 16)], vv, mask=m)
                plsc.store_compressed(minj.at[pl.ds(0, 16)], jj, mask=m)

                def ebody(i, slot):
                    mv = minv[pl.ds(i, 16)][0]
                    mj = minj[pl.ds(i, 16)]
                    c2 = (iota & 0) + ((mv >> 7) & (_CPC - 1))
                    lv = (iota & 0) + (mv & (_LANE - 1))
                    bufv = (iota & 0) + buf
                    for gi in range(4):
                        col = plsc.load_gather(
                            slab5, [bufv, c2, a_vec[gi], s_vec[gi], lv]
                        )
                        rows.at[slot][pl.ds(gi * 16, 16)] = col
                    plsc.store_scatter(jb, [(iota & 0) + slot], mj, mask=lane0)
                    slot = slot + 1

                    @pl.when(slot == 128)
                    def _():
                        flush()

                    return jnp.where(slot == 128, 0, slot)

                return lax.fori_loop(0, cnt, ebody, slot)

            return lax.fori_loop(0, nvr, rbody, slot)

        lax.fori_loop(0, nl, chunk_body, jnp.int32(0))
        flush()

    return k


_sweep = _make_sweep()


def kernel(input, weight):
    wt = weight.T
    idx = input.reshape(_N).astype(jnp.int32)
    out = _sweep(wt, idx)
    return out[:_N, :_D].reshape(_BATCH, _HIST, _D)


# speedup vs baseline: 2.0268x; 1.0193x over previous
import functools

import jax
import jax.numpy as jnp
from jax import lax
from jax.experimental import pallas as pl
from jax.experimental.pallas import tpu as pltpu
from jax.experimental.pallas import tpu_sc as plsc

_BATCH = 1024
_HIST = 20
_D = 64
_N = _BATCH * _HIST
_V = 1000000
_NW = 32
_LANE = 128
_CPC = 2
_CHW = _CPC * _LANE
_NCH = (_V + _CHW - 1) // _CHW
_NOUT = _N + 8
_DUMP = _N


def _make_sweep(V=_V, N=_N):
    mesh = plsc.VectorSubcoreMesh(core_axis_name="c", subcore_axis_name="s")
    nch = (V + _CHW - 1) // _CHW
    tcols = (V + _LANE - 1) // _LANE
    nout = N + 8
    dump = N
    nsub = N // 2560

    @functools.partial(
        pl.kernel,
        mesh=mesh,
        out_type=jax.ShapeDtypeStruct((nout, _LANE), jnp.float32),
        compiler_params=pltpu.CompilerParams(
            use_tc_tiling_on_sc=True, needs_layout_passes=False
        ),
        scratch_types=[
            pltpu.VMEM((2560,), jnp.int32),
            pltpu.VMEM((N + 16,), jnp.int32),
            pltpu.VMEM((N + 16,), jnp.int32),
            pltpu.VMEM((4, _CPC, _D, _LANE), jnp.float32),
            pltpu.VMEM((128, _LANE), jnp.float32),
            pltpu.VMEM((128,), jnp.int32),
            pltpu.VMEM((32,), jnp.int32),
            pltpu.VMEM((32,), jnp.int32),
            pltpu.SemaphoreType.DMA((4,)),
            pltpu.SemaphoreType.DMA,
        ],
    )
    def k(wt, idx, out, idx_st, vlist, jlist, slab, rows, jb, minv, minj,
          sems, ssem):
        t = lax.axis_index("s") * 2 + lax.axis_index("c")
        iota = lax.iota(jnp.int32, 16)
        lane0 = iota == 0
        nl = jnp.where(t < nch - (nch // _NW) * _NW, nch // _NW + 1,
                       nch // _NW)

        wp = jnp.int32(0)
        for sub in range(nsub):
            pltpu.sync_copy(idx.at[pl.ds(sub * 2560, 2560)], idx_st)

            def fbody(r, wp, sub=sub):
                vv = idx_st[pl.ds(r * 16, 16)]
                jj = iota + (sub * 2560) + r * 16
                m = ((vv >> 8) & (_NW - 1)) == t
                plsc.store_compressed(vlist.at[pl.ds(wp, 16)], vv, mask=m)
                plsc.store_compressed(jlist.at[pl.ds(wp, 16)], jj, mask=m)
                return wp + plsc.all_reduce_population_count(m)[0]

            wp = lax.fori_loop(0, 160, fbody, wp)
        vlist[pl.ds(wp, 16)] = jnp.full((16,), -1, jnp.int32)
        jlist[pl.ds(wp, 16)] = jnp.full((16,), dump, jnp.int32)
        nvr = (wp + 15) >> 4

        for g8 in range(8):
            jb[pl.ds(g8 * 16, 16)] = jnp.full((16,), dump, jnp.int32)

        slab5 = slab.reshape(4, _CPC, 8, 8, _LANE)
        a_vec = [(iota + 16 * gi) >> 3 for gi in range(4)]
        s_vec = [(iota + 16 * gi) & 7 for gi in range(4)]

        def fetch(g, buf):
            for c in range(_CPC):
                col = jnp.minimum(g * _CPC + c, tcols - 1)
                col0 = pl.multiple_of(col * _LANE, _LANE)
                pltpu.async_copy(
                    wt.at[:, pl.ds(col0, _LANE)], slab.at[buf, c], sems.at[buf]
                )

        def wait_slab(buf):
            for c in range(_CPC):
                pltpu.make_async_copy(
                    wt.at[:, pl.ds(0, _LANE)], slab.at[buf, c], sems.at[buf]
                ).wait()

        def flush():
            pltpu.async_copy(rows, out.at[jb], ssem).wait()
            for g8 in range(8):
                jb[pl.ds(g8 * 16, 16)] = jnp.full((16,), dump, jnp.int32)

        for p in range(3):
            fetch(p * _NW + t, p)

        def chunk_body(l, slot):
            buf = l & 3
            g = l * _NW + t

            @pl.when(l + 3 < nl)
            def _():
                fetch((l + 3) * _NW + t, (l + 3) & 3)

            wait_slab(buf)

            def rbody(r, slot):
                vv = vlist[pl.ds(r * 16, 16)]
                jj = jlist[pl.ds(r * 16, 16)]
                m = (vv >> 8) == g
                cnt = plsc.all_reduce_population_count(m)[0]
                plsc.store_compressed(minv.at[pl.ds(0, 16)], vv, mask=m)
                plsc.store_compressed(minj.at[pl.ds(0, 16)], jj, mask=m)

                def ebody(i, slot):
                    mv = minv[pl.ds(i, 16)][0]
                    mj = minj[pl.ds(i, 16)]
                    c2 = (iota & 0) + ((mv >> 7) & (_CPC - 1))
                    lv = (iota & 0) + (mv & (_LANE - 1))
                    bufv = (iota & 0) + buf
                    for gi in range(4):
                        col = plsc.load_gather(
                            slab5, [bufv, c2, a_vec[gi], s_vec[gi], lv]
                        )
                        rows.at[slot][pl.ds(gi * 16, 16)] = col
                    plsc.store_scatter(jb, [(iota & 0) + slot], mj, mask=lane0)
                    slot = slot + 1

                    @pl.when(slot == 128)
                    def _():
                        flush()

                    return jnp.where(slot == 128, 0, slot)

                return lax.fori_loop(0, cnt, ebody, slot)

            return lax.fori_loop(0, nvr, rbody, slot)

        lax.fori_loop(0, nl, chunk_body, jnp.int32(0))
        flush()

    return k


_sweep = _make_sweep()


def kernel(input, weight):
    wt = weight.T
    idx = input.reshape(_N).astype(jnp.int32)
    out = _sweep(wt, idx)
    return out[:_N, :_D].reshape(_BATCH, _HIST, _D)


# speedup vs baseline: 2.1835x; 1.0773x over previous
import functools

import jax
import jax.numpy as jnp
from jax import lax
from jax.experimental import pallas as pl
from jax.experimental.pallas import tpu as pltpu
from jax.experimental.pallas import tpu_sc as plsc

_BATCH = 1024
_HIST = 20
_D = 64
_N = _BATCH * _HIST
_V = 1000000
_NW = 32
_LANE = 128
_CPC = 2
_CHW = _CPC * _LANE
_NRING = 4
_MAXL = 128


def _make_sweep(V=_V, N=_N):
    mesh = plsc.VectorSubcoreMesh(core_axis_name="c", subcore_axis_name="s")
    nch = (V + _CHW - 1) // _CHW
    tcols = (V + _LANE - 1) // _LANE
    nout = N + 8
    dump = N
    nsub = N // 2560

    @functools.partial(
        pl.kernel,
        mesh=mesh,
        out_type=jax.ShapeDtypeStruct((nout, _LANE), jnp.float32),
        compiler_params=pltpu.CompilerParams(
            use_tc_tiling_on_sc=True, needs_layout_passes=False
        ),
        scratch_types=[
            pltpu.VMEM((2560,), jnp.int32),
            pltpu.VMEM((N + 16,), jnp.int32),
            pltpu.VMEM((N + 16,), jnp.int32),
            pltpu.VMEM((16 * _MAXL,), jnp.int32),
            pltpu.VMEM((_MAXL + 16,), jnp.int32),
            pltpu.VMEM((_NRING, _CPC, _D, _LANE), jnp.float32),
            pltpu.VMEM((128, _LANE), jnp.float32),
            pltpu.VMEM((128,), jnp.int32),
            pltpu.VMEM((32,), jnp.int32),
            pltpu.VMEM((32,), jnp.int32),
            pltpu.SMEM((_MAXL,), jnp.int32),
            pltpu.SemaphoreType.DMA((_NRING,)),
            pltpu.SemaphoreType.DMA,
        ],
    )
    def k(wt, idx, out, idx_st, gv, gj, hist, incl, slab, rows, jb, stv, stj,
          scur, sems, ssem):
        t = lax.axis_index("s") * 2 + lax.axis_index("c")
        iota = lax.iota(jnp.int32, 16)
        lane0 = iota == 0
        zeros16 = iota & 0
        nl = jnp.where(t < nch - (nch // _NW) * _NW, nch // _NW + 1,
                       nch // _NW)

        for z in range(16 * _MAXL // 256):
            for q in range(16):
                hist[pl.ds(z * 256 + q * 16, 16)] = zeros16
        ones = zeros16 + 1
        for sub in range(nsub):
            pltpu.sync_copy(idx.at[pl.ds(sub * 2560, 2560)], idx_st)

            def hbody(r, c):
                vv = idx_st[pl.ds(r * 16, 16)]
                m = ((vv >> 8) & (_NW - 1)) == t
                lv = vv >> 13
                plsc.addupdate_scatter(hist, [iota * _MAXL + lv], ones, mask=m)
                return c

            lax.fori_loop(0, 160, hbody, jnp.int32(0))

        carry = jnp.int32(0)
        for q in range(_MAXL // 16):
            acc = zeros16
            for kk in range(16):
                acc = acc + hist[pl.ds(kk * _MAXL + q * 16, 16)]
            c = plsc.cumsum(acc) + carry
            incl[pl.ds(q * 16, 16)] = c
            carry = c[15]
        incl[pl.ds(_MAXL, 16)] = zeros16
        def cbody(l, c):
            prev = incl[pl.ds(jnp.maximum(l - 1, 0), 16)][0]
            scur[l] = jnp.where(l == 0, 0, prev)
            return c

        lax.fori_loop(0, _MAXL, cbody, jnp.int32(0))

        for sub in range(nsub):
            pltpu.sync_copy(idx.at[pl.ds(sub * 2560, 2560)], idx_st)

            def pbody(r, c, sub=sub):
                vv = idx_st[pl.ds(r * 16, 16)]
                jj = iota + (sub * 2560) + r * 16
                m = ((vv >> 8) & (_NW - 1)) == t
                cnt = plsc.all_reduce_population_count(m)[0]
                plsc.store_compressed(stv.at[pl.ds(0, 16)], vv, mask=m)
                plsc.store_compressed(stj.at[pl.ds(0, 16)], jj, mask=m)

                def place(i, c):
                    v0 = stv[pl.ds(i, 16)][0]
                    j0 = stj[pl.ds(i, 16)][0]
                    l0 = v0 >> 13
                    p = scur[l0]
                    scur[l0] = p + 1
                    plsc.store_scatter(gv, [zeros16 + p], zeros16 + v0,
                                       mask=lane0)
                    plsc.store_scatter(gj, [zeros16 + p], zeros16 + j0,
                                       mask=lane0)
                    return c

                return lax.fori_loop(0, cnt, place, c)

            lax.fori_loop(0, 160, pbody, jnp.int32(0))

        for g8 in range(8):
            jb[pl.ds(g8 * 16, 16)] = zeros16 + dump

        slab5 = slab.reshape(_NRING, _CPC, 8, 8, _LANE)
        a_vec = [(iota + 16 * gi) >> 3 for gi in range(4)]
        s_vec = [(iota + 16 * gi) & 7 for gi in range(4)]

        def fetch(g, buf):
            for c in range(_CPC):
                col = jnp.minimum(g * _CPC + c, tcols - 1)
                col0 = pl.multiple_of(col * _LANE, _LANE)
                pltpu.async_copy(
                    wt.at[:, pl.ds(col0, _LANE)], slab.at[buf, c], sems.at[buf]
                )

        def wait_slab(buf):
            for c in range(_CPC):
                pltpu.make_async_copy(
                    wt.at[:, pl.ds(0, _LANE)], slab.at[buf, c], sems.at[buf]
                ).wait()

        def flush():
            pltpu.async_copy(rows, out.at[jb], ssem).wait()
            for g8 in range(8):
                jb[pl.ds(g8 * 16, 16)] = zeros16 + dump

        for p in range(_NRING - 1):
            fetch(p * _NW + t, p)

        def chunk_body(l, slot):
            buf = l & (_NRING - 1)

            @pl.when(l + (_NRING - 1) < nl)
            def _():
                fetch((l + (_NRING - 1)) * _NW + t,
                      (l + (_NRING - 1)) & (_NRING - 1))

            wait_slab(buf)
            lo = jnp.where(l == 0, 0,
                           incl[pl.ds(jnp.maximum(l - 1, 0), 16)][0])
            hi = incl[pl.ds(l, 16)][0]
            bufv = zeros16 + buf

            def ebody(i, slot):
                mv0 = gv[pl.ds(i, 16)][0]
                mjv = gj[pl.ds(i, 16)]
                c2 = zeros16 + ((mv0 >> 7) & (_CPC - 1))
                lv = zeros16 + (mv0 & (_LANE - 1))
                for gi in range(4):
                    col = plsc.load_gather(
                        slab5, [bufv, c2, a_vec[gi], s_vec[gi], lv]
                    )
                    rows.at[slot][pl.ds(gi * 16, 16)] = col
                plsc.store_scatter(jb, [zeros16 + slot], mjv, mask=lane0)
                slot = slot + 1

                @pl.when(slot == 128)
                def _():
                    flush()

                return jnp.where(slot == 128, 0, slot)

            return lax.fori_loop(lo, hi, ebody, slot)

        lax.fori_loop(0, nl, chunk_body, jnp.int32(0))
        flush()

    return k


_sweep = _make_sweep()


def kernel(input, weight):
    wt = weight.T
    idx = input.reshape(_N).astype(jnp.int32)
    out = _sweep(wt, idx)
    return out[:_N, :_D].reshape(_BATCH, _HIST, _D)
